# Initial kernel scaffold; baseline (speedup 1.0000x reference)
import jax
import jax.numpy as jnp
from jax.experimental import pallas as pl


def kernel(node_attrs, node_feats, edge_attrs, edge_feats, edge_index, W_lin1, W_mlp0, W_mlp1, W_mlp2, W_mlp3, W_lin2, W_skip):
    raise NotImplementedError("write your pallas kernel here")


# speedup vs baseline: 2.0841x; 2.0841x over previous
import functools

import jax
import jax.numpy as jnp
import numpy as np
from jax import lax
from jax.experimental import pallas as pl
from jax.experimental.pallas import tpu as pltpu
from jax.experimental.pallas import tpu_sc as plsc

N = 10000
E = 320000
D = 128
A = 10
R = 8
H = 64
AVG_NUM_NEIGHBORS = 32.0

NC = 2
NS = 16
NW = NC * NS
EPW = E // NW
C = 80
NCHUNK = EPW // C
NPAD = 10240
NPS = NPAD // NS

BN = 1000
BE = 2000


def _wc_body(w0, w1, w2, w3, out):
    h = jnp.dot(w0[...], w1[...], preferred_element_type=jnp.float32)
    h = jnp.dot(h, w2[...], preferred_element_type=jnp.float32)
    h = jnp.dot(h, w3[...], preferred_element_type=jnp.float32)
    out[...] = h * np.float32(1.0 / np.sqrt(R * H * H * H))


_wc_call = pl.pallas_call(
    _wc_body,
    out_shape=jax.ShapeDtypeStruct((R, D), jnp.float32),
)


def _xsc_body(nf, na, w1, wsk, x_out, sc_out):
    x = jnp.dot(nf[...], w1[...], preferred_element_type=jnp.float32)
    x = x * np.float32(1.0 / np.sqrt(D))
    x_out[...] = x
    acc = jnp.zeros((BN, D), jnp.float32)
    for v in range(A):
        acc = acc + na[:, v][:, None] * jnp.dot(
            x, wsk[v], preferred_element_type=jnp.float32)
    sc_out[...] = acc * np.float32(1.0 / np.sqrt(D * A))


_xsc_call = pl.pallas_call(
    _xsc_body,
    grid=(N // BN,),
    in_specs=[
        pl.BlockSpec((BN, D), lambda i: (i, 0)),
        pl.BlockSpec((BN, A), lambda i: (i, 0)),
        pl.BlockSpec((D, D), lambda i: (0, 0)),
        pl.BlockSpec((A, D, D), lambda i: (0, 0, 0)),
    ],
    out_specs=[
        pl.BlockSpec((BN, D), lambda i: (i, 0)),
        pl.BlockSpec((BN, D), lambda i: (i, 0)),
    ],
    out_shape=[
        jax.ShapeDtypeStruct((N, D), jnp.float32),
        jax.ShapeDtypeStruct((N, D), jnp.float32),
    ],
)


def _we_body(ef, ea, wc, out):
    out[...] = jnp.dot(ef[...] * ea[...], wc[...],
                       preferred_element_type=jnp.float32)


_we_call = pl.pallas_call(
    _we_body,
    grid=(E // BE,),
    in_specs=[
        pl.BlockSpec((BE, R), lambda i: (i, 0)),
        pl.BlockSpec((BE, 1), lambda i: (i, 0)),
        pl.BlockSpec((R, D), lambda i: (0, 0)),
    ],
    out_specs=pl.BlockSpec((BE, D), lambda i: (i, 0)),
    out_shape=jax.ShapeDtypeStruct((E, D), jnp.float32),
)


_sc_mesh = plsc.VectorSubcoreMesh(
    core_axis_name="c", subcore_axis_name="s", num_cores=NC, num_subcores=NS)


@functools.partial(
    pl.kernel,
    out_type=jax.ShapeDtypeStruct((NC * NPAD, D), jnp.float32),
    mesh=_sc_mesh,
    scratch_types=[
        pltpu.VMEM((C,), jnp.int32),
        pltpu.VMEM((C,), jnp.int32),
        pltpu.VMEM((C, D), jnp.float32),
        pltpu.VMEM((C, D), jnp.float32),
        pltpu.VMEM_SHARED((NPAD, D), jnp.float32),
        pltpu.SemaphoreType.DMA,
    ],
)
def _edge_call(x_hbm, we_hbm, send_hbm, recv_hbm, zero_hbm, out_hbm,
               sidx, ridx, rows, wrows, acc, sem):
    c = lax.axis_index("c")
    s = lax.axis_index("s")
    w = s * NC + c
    pltpu.sync_copy(zero_hbm.at[pl.ds(s * NPS, NPS)],
                    acc.at[pl.ds(s * NPS, NPS)])
    plsc.subcore_barrier()

    ebase = w * EPW

    def chunk(i, carry):
        base = ebase + i * C
        pltpu.sync_copy(send_hbm.at[pl.ds(base, C)], sidx)
        pltpu.sync_copy(recv_hbm.at[pl.ds(base, C)], ridx)
        pltpu.sync_copy(we_hbm.at[pl.ds(base, C), :], wrows)
        pltpu.async_copy(x_hbm.at[sidx], rows, sem).wait()

        def rowmul(r, rcarry):
            for j in range(D // 16):
                sl = pl.ds(j * 16, 16)
                rows[r, sl] = rows[r, sl] * wrows[r, sl]
            return rcarry

        lax.fori_loop(0, C, rowmul, 0)
        pltpu.sync_copy(rows, acc.at[ridx], add=True)
        return carry

    lax.fori_loop(0, NCHUNK, chunk, 0)
    plsc.subcore_barrier()
    pltpu.sync_copy(acc.at[pl.ds(s * NPS, NPS)],
                    out_hbm.at[pl.ds(c * NPAD + s * NPS, NPS)])


def _post_body(p0, p1, w2, out):
    m = jnp.dot(p0[...] + p1[...], w2[...], preferred_element_type=jnp.float32)
    out[...] = m * np.float32(1.0 / (np.sqrt(D) * AVG_NUM_NEIGHBORS))


_post_call = pl.pallas_call(
    _post_body,
    grid=(N // BN,),
    in_specs=[
        pl.BlockSpec((BN, D), lambda i: (i, 0)),
        pl.BlockSpec((BN, D), lambda i: (i, 0)),
        pl.BlockSpec((D, D), lambda i: (0, 0)),
    ],
    out_specs=pl.BlockSpec((BN, D), lambda i: (i, 0)),
    out_shape=jax.ShapeDtypeStruct((N, D), jnp.float32),
)


def kernel(node_attrs, node_feats, edge_attrs, edge_feats, edge_index,
           W_lin1, W_mlp0, W_mlp1, W_mlp2, W_mlp3, W_lin2, W_skip):
    sender = edge_index[0]
    receiver = edge_index[1]

    wc = _wc_call(W_mlp0, W_mlp1, W_mlp2, W_mlp3)
    wt = jnp.transpose(W_skip, (1, 0, 2))
    x, sc = _xsc_call(node_feats, node_attrs, W_lin1, wt)
    we = _we_call(edge_feats, edge_attrs, wc)

    zeros = jnp.zeros((NPAD, D), jnp.float32)
    partials = _edge_call(x, we, sender, receiver, zeros)
    message = _post_call(partials[:N], partials[NPAD:NPAD + N], W_lin2)
    return message.reshape(N, D, 1), sc
